# Initial kernel scaffold; baseline (speedup 1.0000x reference)
import functools

import jax
import jax.numpy as jnp
from jax import lax
from jax.experimental import pallas as pl
from jax.experimental.pallas import tpu as pltpu
from jax.experimental.pallas import tpu_sc as plsc

NC = 2
NS = 16
NW = NC * NS
KB = 128


def _make_deg_kernel(n, nb):
    per_w = nb // NW
    rem = nb - per_w * NW
    n_pad = NS * ((n + NS - 1) // NS // 8 * 8 + 8)
    chunk = n_pad // NS

    mesh = plsc.VectorSubcoreMesh(core_axis_name="c", subcore_axis_name="s")

    @functools.partial(
        pl.kernel,
        mesh=mesh,
        out_type=jax.ShapeDtypeStruct((NC, n), jnp.float32),
        scratch_types=[
            pltpu.VMEM((KB,), jnp.int32),
            pltpu.VMEM((KB,), jnp.float32),
            pltpu.VMEM((chunk,), jnp.float32),
            pltpu.VMEM_SHARED((n_pad,), jnp.float32),
        ],
    )
    def deg_kernel(dst2_hbm, out_hbm, idx_v, ones_v, z_v, hist):
        c = lax.axis_index("c")
        s = lax.axis_index("s")
        w = s * NC + c

        def fill(i, _):
            ones_v[pl.ds(i * 16, 16)] = jnp.full((16,), 1.0, jnp.float32)
            return _

        lax.fori_loop(0, KB // 16, fill, None)

        def zfill(i, _):
            z_v[pl.ds(i * 16, 16)] = jnp.zeros((16,), jnp.float32)
            return _

        lax.fori_loop(0, chunk // 16, zfill, None)
        pltpu.sync_copy(z_v, hist.at[pl.ds(s * chunk, chunk)])
        plsc.subcore_barrier()

        def body(i, _):
            bj = w * per_w + i
            pltpu.sync_copy(dst2_hbm.at[bj], idx_v)
            pltpu.sync_copy(ones_v, hist.at[idx_v], add=True)
            return _

        lax.fori_loop(0, per_w, body, None)
        if rem:
            @pl.when(w < rem)
            def _():
                bj = NW * per_w + w
                pltpu.sync_copy(dst2_hbm.at[bj], idx_v)
                pltpu.sync_copy(ones_v, hist.at[idx_v], add=True)

        plsc.subcore_barrier()

        @pl.when(s == 0)
        def _():
            pltpu.sync_copy(hist.at[pl.ds(0, n)], out_hbm.at[c])

    return deg_kernel


def _make_agg_kernel(n, d, nb):
    per_w = nb // NW
    rem = nb - per_w * NW
    rows_per_tile = (n + NS - 1) // NS

    mesh = plsc.VectorSubcoreMesh(core_axis_name="c", subcore_axis_name="s")

    @functools.partial(
        pl.kernel,
        mesh=mesh,
        out_type=jax.ShapeDtypeStruct((NC, n, d), jnp.float32),
        scratch_types=[
            pltpu.VMEM((KB,), jnp.int32),
            pltpu.VMEM((KB,), jnp.int32),
            pltpu.VMEM((KB, d), jnp.float32),
            pltpu.VMEM((KB, d), jnp.float32),
            pltpu.VMEM_SHARED((NS * ((n + NS - 1) // NS), d), jnp.float32),
            pltpu.SemaphoreType.DMA,
        ],
    )
    def agg_kernel(hs_hbm, src2_hbm, dst2_hbm, out_hbm, idx_s, idx_d, rows_v,
                   z_v, acc, sem):
        c = lax.axis_index("c")
        s = lax.axis_index("s")
        w = s * NC + c
        rows_per_tile = acc.shape[0] // NS

        def zrow(i, _):
            for g in range(d // 16):
                z_v[i, pl.ds(g * 16, 16)] = jnp.zeros((16,), jnp.float32)
            return _

        lax.fori_loop(0, KB, zrow, None)
        r0 = s * rows_per_tile
        off = 0
        while off < rows_per_tile:
            sz = min(KB, rows_per_tile - off)
            pltpu.sync_copy(z_v.at[pl.ds(0, sz)], acc.at[pl.ds(r0 + off, sz)])
            off += sz
        plsc.subcore_barrier()

        def do_block(bj):
            pltpu.sync_copy(src2_hbm.at[bj], idx_s)
            pltpu.sync_copy(dst2_hbm.at[bj], idx_d)
            pltpu.async_copy(hs_hbm.at[idx_s], rows_v, sem).wait()
            pltpu.sync_copy(rows_v, acc.at[idx_d], add=True)

        def body(i, _):
            do_block(w * per_w + i)
            return _

        lax.fori_loop(0, per_w, body, None)
        if rem:
            @pl.when(w < rem)
            def _():
                do_block(NW * per_w + w)

        plsc.subcore_barrier()
        lo = s * rows_per_tile
        sz = min(n - s * rows_per_tile, rows_per_tile)
        pltpu.sync_copy(acc.at[pl.ds(lo, sz)],
                        out_hbm.at[c, pl.ds(lo, sz)])

    return agg_kernel


def _scale_mm(degT_ref, x_ref, w_ref, hs_ref, dis_ref):
    dis = lax.rsqrt(1.0 + degT_ref[:, 0:1] + degT_ref[:, 1:2])
    hs_ref[...] = dis * jnp.dot(x_ref[...], w_ref[...],
                                preferred_element_type=jnp.float32)
    dis_ref[...] = dis


def _mid_layer(agg_ref, hs_ref, dis_ref, b1_ref, w2_ref, o_ref):
    a = agg_ref[0] + agg_ref[1] + hs_ref[...]
    h1 = jnp.tanh(dis_ref[...] * a + b1_ref[...])
    o_ref[...] = dis_ref[...] * jnp.dot(h1, w2_ref[...],
                                        preferred_element_type=jnp.float32)


def _final_head(agg_ref, hs_ref, dis_ref, b2_ref, w3_ref, b3_ref, w4_ref,
                b4_ref, o_ref):
    a = agg_ref[0] + agg_ref[1] + hs_ref[...]
    h2 = jnp.tanh(dis_ref[...] * a + b2_ref[...])
    h3 = jnp.tanh(jnp.dot(h2, w3_ref[...], preferred_element_type=jnp.float32)
                  + b3_ref[...])
    o_ref[...] = jnp.dot(h3, w4_ref[...],
                         preferred_element_type=jnp.float32) + b4_ref[...]


def kernel(x, edge_index, W1, b1, W2, b2, W3, b3, W4, b4):
    n, d = x.shape
    e = edge_index.shape[1]
    nb = e // KB
    assert nb * KB == e

    src2 = edge_index[0].reshape(nb, KB)
    dst2 = edge_index[1].reshape(nb, KB)

    deg_kernel = _make_deg_kernel(n, nb)
    agg_kernel = _make_agg_kernel(n, d, nb)

    degp = deg_kernel(dst2)
    degT = degp.T

    bm = 1000
    grid = (n // bm,)
    f32 = jnp.float32

    hs1, dis = pl.pallas_call(
        _scale_mm,
        grid=grid,
        in_specs=[
            pl.BlockSpec((bm, 2), lambda i: (i, 0)),
            pl.BlockSpec((bm, d), lambda i: (i, 0)),
            pl.BlockSpec((d, d), lambda i: (0, 0)),
        ],
        out_specs=[
            pl.BlockSpec((bm, d), lambda i: (i, 0)),
            pl.BlockSpec((bm, 1), lambda i: (i, 0)),
        ],
        out_shape=[
            jax.ShapeDtypeStruct((n, d), f32),
            jax.ShapeDtypeStruct((n, 1), f32),
        ],
    )(degT, x, W1)

    agg1 = agg_kernel(hs1, src2, dst2)

    hs2 = pl.pallas_call(
        _mid_layer,
        grid=grid,
        in_specs=[
            pl.BlockSpec((2, bm, d), lambda i: (0, i, 0)),
            pl.BlockSpec((bm, d), lambda i: (i, 0)),
            pl.BlockSpec((bm, 1), lambda i: (i, 0)),
            pl.BlockSpec((d,), lambda i: (0,)),
            pl.BlockSpec((d, d), lambda i: (0, 0)),
        ],
        out_specs=pl.BlockSpec((bm, d), lambda i: (i, 0)),
        out_shape=jax.ShapeDtypeStruct((n, d), f32),
    )(agg1, hs1, dis, b1, W2)

    agg2 = agg_kernel(hs2, src2, dst2)

    d3 = W3.shape[1]
    out = pl.pallas_call(
        _final_head,
        grid=grid,
        in_specs=[
            pl.BlockSpec((2, bm, d), lambda i: (0, i, 0)),
            pl.BlockSpec((bm, d), lambda i: (i, 0)),
            pl.BlockSpec((bm, 1), lambda i: (i, 0)),
            pl.BlockSpec((d,), lambda i: (0,)),
            pl.BlockSpec((d, d3), lambda i: (0, 0)),
            pl.BlockSpec((d3,), lambda i: (0,)),
            pl.BlockSpec((d3, 1), lambda i: (0, 0)),
            pl.BlockSpec((1,), lambda i: (0,)),
        ],
        out_specs=pl.BlockSpec((bm, 1), lambda i: (i, 0)),
        out_shape=jax.ShapeDtypeStruct((n, 1), f32),
    )(agg2, hs2, dis, b2, W3, b3, W4, b4)

    return out


# speedup vs baseline: 15.9512x; 15.9512x over previous
import functools

import jax
import jax.numpy as jnp
from jax import lax
from jax.experimental import pallas as pl
from jax.experimental.pallas import tpu as pltpu
from jax.experimental.pallas import tpu_sc as plsc

NC = 2
NS = 16
NW = NC * NS
KB = 128


def _make_deg_kernel(n, nb):
    per_w = nb // NW
    rem = nb - per_w * NW
    chunk = -(-((n + NS - 1) // NS) // 128) * 128
    n_pad = NS * chunk

    mesh = plsc.VectorSubcoreMesh(core_axis_name="c", subcore_axis_name="s")

    @functools.partial(
        pl.kernel,
        mesh=mesh,
        out_type=jax.ShapeDtypeStruct((NC, n_pad), jnp.float32),
        scratch_types=[
            pltpu.VMEM((KB,), jnp.int32),
            pltpu.VMEM((KB,), jnp.float32),
            pltpu.VMEM((chunk,), jnp.float32),
            pltpu.VMEM_SHARED((n_pad,), jnp.float32),
        ],
    )
    def deg_kernel(dst2_hbm, out_hbm, idx_v, ones_v, z_v, hist):
        c = lax.axis_index("c")
        s = lax.axis_index("s")
        w = s * NC + c

        def fill(i, _):
            ones_v[pl.ds(i * 16, 16)] = jnp.full((16,), 1.0, jnp.float32)
            return _

        lax.fori_loop(0, KB // 16, fill, None)

        def zfill(i, _):
            z_v[pl.ds(i * 16, 16)] = jnp.zeros((16,), jnp.float32)
            return _

        lax.fori_loop(0, chunk // 16, zfill, None)
        pltpu.sync_copy(z_v, hist.at[pl.ds(s * chunk, chunk)])
        plsc.subcore_barrier()

        def body(i, _):
            bj = w * per_w + i
            pltpu.sync_copy(dst2_hbm.at[bj], idx_v)
            pltpu.sync_copy(ones_v, hist.at[idx_v], add=True)
            return _

        lax.fori_loop(0, per_w, body, None)
        if rem:
            @pl.when(w < rem)
            def _():
                bj = NW * per_w + w
                pltpu.sync_copy(dst2_hbm.at[bj], idx_v)
                pltpu.sync_copy(ones_v, hist.at[idx_v], add=True)

        plsc.subcore_barrier()
        pltpu.sync_copy(hist.at[pl.ds(s * chunk, chunk)],
                        out_hbm.at[c, pl.ds(s * chunk, chunk)])

    return deg_kernel


def _make_agg_kernel(n, d, nb):
    per_w = nb // NW
    rem = nb - per_w * NW
    rows_per_tile = -(-((n + NS - 1) // NS) // 8) * 8
    n_pad = NS * rows_per_tile

    mesh = plsc.VectorSubcoreMesh(core_axis_name="c", subcore_axis_name="s")

    @functools.partial(
        pl.kernel,
        mesh=mesh,
        out_type=jax.ShapeDtypeStruct((NC, n_pad, d), jnp.float32),
        scratch_types=[
            pltpu.VMEM((KB,), jnp.int32),
            pltpu.VMEM((KB,), jnp.int32),
            pltpu.VMEM((KB, d), jnp.float32),
            pltpu.VMEM((KB, d), jnp.float32),
            pltpu.VMEM_SHARED((n_pad, d), jnp.float32),
            pltpu.SemaphoreType.DMA,
        ],
    )
    def agg_kernel(hs_hbm, src2_hbm, dst2_hbm, out_hbm, idx_s, idx_d, rows_v,
                   z_v, acc, sem):
        c = lax.axis_index("c")
        s = lax.axis_index("s")
        w = s * NC + c

        def zrow(i, _):
            for g in range(d // 16):
                z_v[i, pl.ds(g * 16, 16)] = jnp.zeros((16,), jnp.float32)
            return _

        lax.fori_loop(0, KB, zrow, None)
        r0 = s * rows_per_tile
        off = 0
        while off < rows_per_tile:
            sz = min(KB, rows_per_tile - off)
            pltpu.sync_copy(z_v.at[pl.ds(0, sz)], acc.at[pl.ds(r0 + off, sz)])
            off += sz
        plsc.subcore_barrier()

        def do_block(bj):
            pltpu.sync_copy(src2_hbm.at[bj], idx_s)
            pltpu.sync_copy(dst2_hbm.at[bj], idx_d)
            pltpu.async_copy(hs_hbm.at[idx_s], rows_v, sem).wait()
            pltpu.sync_copy(rows_v, acc.at[idx_d], add=True)

        def body(i, _):
            do_block(w * per_w + i)
            return _

        lax.fori_loop(0, per_w, body, None)
        if rem:
            @pl.when(w < rem)
            def _():
                do_block(NW * per_w + w)

        plsc.subcore_barrier()
        lo = s * rows_per_tile
        pltpu.sync_copy(acc.at[pl.ds(lo, rows_per_tile)],
                        out_hbm.at[c, pl.ds(lo, rows_per_tile)])

    return agg_kernel


def _scale_mm(degT_ref, x_ref, w_ref, hs_ref, dis_ref):
    dis = lax.rsqrt(1.0 + degT_ref[:, 0:1] + degT_ref[:, 1:2])
    hs_ref[...] = dis * jnp.dot(x_ref[...], w_ref[...],
                                preferred_element_type=jnp.float32)
    dis_ref[...] = dis


def _mid_layer(agg_ref, hs_ref, dis_ref, b1_ref, w2_ref, o_ref):
    a = agg_ref[0] + agg_ref[1] + hs_ref[...]
    h1 = jnp.tanh(dis_ref[...] * a + b1_ref[...])
    o_ref[...] = dis_ref[...] * jnp.dot(h1, w2_ref[...],
                                        preferred_element_type=jnp.float32)


def _final_head(agg_ref, hs_ref, dis_ref, b2_ref, w3_ref, b3_ref, w4_ref,
                b4_ref, o_ref):
    a = agg_ref[0] + agg_ref[1] + hs_ref[...]
    h2 = jnp.tanh(dis_ref[...] * a + b2_ref[...])
    h3 = jnp.tanh(jnp.dot(h2, w3_ref[...], preferred_element_type=jnp.float32)
                  + b3_ref[...])
    o_ref[...] = jnp.dot(h3, w4_ref[...],
                         preferred_element_type=jnp.float32) + b4_ref[...]


def kernel(x, edge_index, W1, b1, W2, b2, W3, b3, W4, b4):
    n, d = x.shape
    e = edge_index.shape[1]
    nb = e // KB
    assert nb * KB == e

    src2 = edge_index[0].reshape(nb, KB)
    dst2 = edge_index[1].reshape(nb, KB)

    deg_kernel = _make_deg_kernel(n, nb)
    agg_kernel = _make_agg_kernel(n, d, nb)

    degp = deg_kernel(dst2)[:, :n]
    degT = degp.T

    bm = 1000
    grid = (n // bm,)
    f32 = jnp.float32

    hs1, dis = pl.pallas_call(
        _scale_mm,
        grid=grid,
        in_specs=[
            pl.BlockSpec((bm, 2), lambda i: (i, 0)),
            pl.BlockSpec((bm, d), lambda i: (i, 0)),
            pl.BlockSpec((d, d), lambda i: (0, 0)),
        ],
        out_specs=[
            pl.BlockSpec((bm, d), lambda i: (i, 0)),
            pl.BlockSpec((bm, 1), lambda i: (i, 0)),
        ],
        out_shape=[
            jax.ShapeDtypeStruct((n, d), f32),
            jax.ShapeDtypeStruct((n, 1), f32),
        ],
    )(degT, x, W1)

    agg1 = agg_kernel(hs1, src2, dst2)[:, :n]

    hs2 = pl.pallas_call(
        _mid_layer,
        grid=grid,
        in_specs=[
            pl.BlockSpec((2, bm, d), lambda i: (0, i, 0)),
            pl.BlockSpec((bm, d), lambda i: (i, 0)),
            pl.BlockSpec((bm, 1), lambda i: (i, 0)),
            pl.BlockSpec((d,), lambda i: (0,)),
            pl.BlockSpec((d, d), lambda i: (0, 0)),
        ],
        out_specs=pl.BlockSpec((bm, d), lambda i: (i, 0)),
        out_shape=jax.ShapeDtypeStruct((n, d), f32),
    )(agg1, hs1, dis, b1, W2)

    agg2 = agg_kernel(hs2, src2, dst2)[:, :n]

    d3 = W3.shape[1]
    out = pl.pallas_call(
        _final_head,
        grid=grid,
        in_specs=[
            pl.BlockSpec((2, bm, d), lambda i: (0, i, 0)),
            pl.BlockSpec((bm, d), lambda i: (i, 0)),
            pl.BlockSpec((bm, 1), lambda i: (i, 0)),
            pl.BlockSpec((d,), lambda i: (0,)),
            pl.BlockSpec((d, d3), lambda i: (0, 0)),
            pl.BlockSpec((d3,), lambda i: (0,)),
            pl.BlockSpec((d3, 1), lambda i: (0, 0)),
            pl.BlockSpec((1,), lambda i: (0,)),
        ],
        out_specs=pl.BlockSpec((bm, 1), lambda i: (i, 0)),
        out_shape=jax.ShapeDtypeStruct((n, 1), f32),
    )(agg2, hs2, dis, b2, W3, b3, W4, b4)

    return out


# speedup vs baseline: 21.3450x; 1.3381x over previous
import functools

import jax
import jax.numpy as jnp
from jax import lax
from jax.experimental import pallas as pl
from jax.experimental.pallas import tpu as pltpu
from jax.experimental.pallas import tpu_sc as plsc

NC = 2
NS = 16
NW = NC * NS
KB = 128
UNROLL = 6


def _make_deg_kernel(n, nb):
    per_w = nb // NW
    rem = nb - per_w * NW
    chunk = -(-((n + NS - 1) // NS) // 128) * 128
    n_pad = NS * chunk

    mesh = plsc.VectorSubcoreMesh(core_axis_name="c", subcore_axis_name="s")

    @functools.partial(
        pl.kernel,
        mesh=mesh,
        out_type=jax.ShapeDtypeStruct((NC, n_pad), jnp.float32),
        scratch_types=[
            pltpu.VMEM((KB,), jnp.int32),
            pltpu.VMEM((KB,), jnp.float32),
            pltpu.VMEM((chunk,), jnp.float32),
            pltpu.VMEM_SHARED((n_pad,), jnp.float32),
        ],
    )
    def deg_kernel(dst2_hbm, out_hbm, idx_v, ones_v, z_v, hist):
        c = lax.axis_index("c")
        s = lax.axis_index("s")
        w = s * NC + c

        def fill(i, _):
            ones_v[pl.ds(i * 16, 16)] = jnp.full((16,), 1.0, jnp.float32)
            return _

        lax.fori_loop(0, KB // 16, fill, None)

        def zfill(i, _):
            z_v[pl.ds(i * 16, 16)] = jnp.zeros((16,), jnp.float32)
            return _

        lax.fori_loop(0, chunk // 16, zfill, None)
        pltpu.sync_copy(z_v, hist.at[pl.ds(s * chunk, chunk)])
        plsc.subcore_barrier()

        def body(i, _):
            bj = w * per_w + i
            pltpu.sync_copy(dst2_hbm.at[bj], idx_v)
            pltpu.sync_copy(ones_v, hist.at[idx_v], add=True)
            return _

        lax.fori_loop(0, per_w, body, None)
        if rem:
            @pl.when(w < rem)
            def _():
                bj = NW * per_w + w
                pltpu.sync_copy(dst2_hbm.at[bj], idx_v)
                pltpu.sync_copy(ones_v, hist.at[idx_v], add=True)

        plsc.subcore_barrier()
        pltpu.sync_copy(hist.at[pl.ds(s * chunk, chunk)],
                        out_hbm.at[c, pl.ds(s * chunk, chunk)])

    return deg_kernel


def _make_agg_kernel(n, d, nb):
    per_w = nb // NW
    rem = nb - per_w * NW
    rows_per_tile = -(-((n + NS - 1) // NS) // 8) * 8
    n_pad = NS * rows_per_tile

    mesh = plsc.VectorSubcoreMesh(core_axis_name="c", subcore_axis_name="s")

    @functools.partial(
        pl.kernel,
        mesh=mesh,
        out_type=jax.ShapeDtypeStruct((NC, n_pad, d), jnp.float32),
        scratch_types=[
            pltpu.VMEM((KB,), jnp.int32),
            pltpu.VMEM((KB,), jnp.int32),
            pltpu.VMEM((KB,), jnp.int32),
            pltpu.VMEM((KB,), jnp.int32),
            pltpu.VMEM((KB, d), jnp.float32),
            pltpu.VMEM((KB, d), jnp.float32),
            pltpu.VMEM((KB, d), jnp.float32),
            pltpu.VMEM_SHARED((n_pad, d), jnp.float32),
            pltpu.SemaphoreType.DMA,
            pltpu.SemaphoreType.DMA,
            pltpu.SemaphoreType.DMA,
            pltpu.SemaphoreType.DMA,
            pltpu.SemaphoreType.DMA,
            pltpu.SemaphoreType.DMA,
        ],
    )
    def agg_kernel(hs_hbm, src2_hbm, dst2_hbm, out_hbm, idx_s, idx_d, idx_s2,
                   idx_d2, rows_v, rows_v2, z_v, acc, sem, sem2, sia, sib,
                   sic, sid):
        c = lax.axis_index("c")
        s = lax.axis_index("s")
        w = s * NC + c

        def zrow(i, _):
            for g in range(d // 16):
                z_v[i, pl.ds(g * 16, 16)] = jnp.zeros((16,), jnp.float32)
            return _

        lax.fori_loop(0, KB, zrow, None)
        r0 = s * rows_per_tile
        off = 0
        while off < rows_per_tile:
            sz = min(KB, rows_per_tile - off)
            pltpu.sync_copy(z_v.at[pl.ds(0, sz)], acc.at[pl.ds(r0 + off, sz)])
            off += sz
        plsc.subcore_barrier()

        def do_block(bj):
            pltpu.sync_copy(src2_hbm.at[bj], idx_s)
            pltpu.sync_copy(dst2_hbm.at[bj], idx_d)
            pltpu.async_copy(hs_hbm.at[idx_s], rows_v, sem).wait()
            pltpu.sync_copy(rows_v, acc.at[idx_d], add=True)

        def body(i, _):
            b0 = w * per_w + 2 * i
            b1 = b0 + 1
            d0 = pltpu.async_copy(src2_hbm.at[b0], idx_s, sia)
            d0b = pltpu.async_copy(dst2_hbm.at[b0], idx_d, sib)
            d1 = pltpu.async_copy(src2_hbm.at[b1], idx_s2, sic)
            d1b = pltpu.async_copy(dst2_hbm.at[b1], idx_d2, sid)
            d0.wait()
            g0 = pltpu.async_copy(hs_hbm.at[idx_s], rows_v, sem)
            d1.wait()
            g1 = pltpu.async_copy(hs_hbm.at[idx_s2], rows_v2, sem2)
            d0b.wait()
            g0.wait()
            pltpu.sync_copy(rows_v, acc.at[idx_d], add=True)
            d1b.wait()
            g1.wait()
            pltpu.sync_copy(rows_v2, acc.at[idx_d2], add=True)
            return _

        lax.fori_loop(0, per_w // 2, body, None)
        for t in range(per_w - 2 * (per_w // 2)):
            do_block(w * per_w + 2 * (per_w // 2) + t)
        if rem:
            @pl.when(w < rem)
            def _():
                do_block(NW * per_w + w)

        plsc.subcore_barrier()
        lo = s * rows_per_tile
        pltpu.sync_copy(acc.at[pl.ds(lo, rows_per_tile)],
                        out_hbm.at[c, pl.ds(lo, rows_per_tile)])

    return agg_kernel


def _scale_mm(degT_ref, x_ref, w_ref, hs_ref, dis_ref):
    dis = lax.rsqrt(1.0 + degT_ref[:, 0:1] + degT_ref[:, 1:2])
    hs_ref[...] = dis * jnp.dot(x_ref[...], w_ref[...],
                                preferred_element_type=jnp.float32)
    dis_ref[...] = dis


def _mid_layer(agg_ref, hs_ref, dis_ref, b1_ref, w2_ref, o_ref):
    a = agg_ref[0] + agg_ref[1] + hs_ref[...]
    h1 = jnp.tanh(dis_ref[...] * a + b1_ref[...])
    o_ref[...] = dis_ref[...] * jnp.dot(h1, w2_ref[...],
                                        preferred_element_type=jnp.float32)


def _final_head(agg_ref, hs_ref, dis_ref, b2_ref, w3_ref, b3_ref, w4_ref,
                b4_ref, o_ref):
    a = agg_ref[0] + agg_ref[1] + hs_ref[...]
    h2 = jnp.tanh(dis_ref[...] * a + b2_ref[...])
    h3 = jnp.tanh(jnp.dot(h2, w3_ref[...], preferred_element_type=jnp.float32)
                  + b3_ref[...])
    o_ref[...] = jnp.dot(h3, w4_ref[...],
                         preferred_element_type=jnp.float32) + b4_ref[...]


def kernel(x, edge_index, W1, b1, W2, b2, W3, b3, W4, b4):
    n, d = x.shape
    e = edge_index.shape[1]
    nb = e // KB
    assert nb * KB == e

    src2 = edge_index[0].reshape(nb, KB)
    dst2 = edge_index[1].reshape(nb, KB)

    deg_kernel = _make_deg_kernel(n, nb)
    agg_kernel = _make_agg_kernel(n, d, nb)

    degp = deg_kernel(dst2)[:, :n]
    degT = degp.T

    bm = 1000
    grid = (n // bm,)
    f32 = jnp.float32

    hs1, dis = pl.pallas_call(
        _scale_mm,
        grid=grid,
        in_specs=[
            pl.BlockSpec((bm, 2), lambda i: (i, 0)),
            pl.BlockSpec((bm, d), lambda i: (i, 0)),
            pl.BlockSpec((d, d), lambda i: (0, 0)),
        ],
        out_specs=[
            pl.BlockSpec((bm, d), lambda i: (i, 0)),
            pl.BlockSpec((bm, 1), lambda i: (i, 0)),
        ],
        out_shape=[
            jax.ShapeDtypeStruct((n, d), f32),
            jax.ShapeDtypeStruct((n, 1), f32),
        ],
    )(degT, x, W1)

    agg1 = agg_kernel(hs1, src2, dst2)[:, :n]

    hs2 = pl.pallas_call(
        _mid_layer,
        grid=grid,
        in_specs=[
            pl.BlockSpec((2, bm, d), lambda i: (0, i, 0)),
            pl.BlockSpec((bm, d), lambda i: (i, 0)),
            pl.BlockSpec((bm, 1), lambda i: (i, 0)),
            pl.BlockSpec((d,), lambda i: (0,)),
            pl.BlockSpec((d, d), lambda i: (0, 0)),
        ],
        out_specs=pl.BlockSpec((bm, d), lambda i: (i, 0)),
        out_shape=jax.ShapeDtypeStruct((n, d), f32),
    )(agg1, hs1, dis, b1, W2)

    agg2 = agg_kernel(hs2, src2, dst2)[:, :n]

    d3 = W3.shape[1]
    out = pl.pallas_call(
        _final_head,
        grid=grid,
        in_specs=[
            pl.BlockSpec((2, bm, d), lambda i: (0, i, 0)),
            pl.BlockSpec((bm, d), lambda i: (i, 0)),
            pl.BlockSpec((bm, 1), lambda i: (i, 0)),
            pl.BlockSpec((d,), lambda i: (0,)),
            pl.BlockSpec((d, d3), lambda i: (0, 0)),
            pl.BlockSpec((d3,), lambda i: (0,)),
            pl.BlockSpec((d3, 1), lambda i: (0, 0)),
            pl.BlockSpec((1,), lambda i: (0,)),
        ],
        out_specs=pl.BlockSpec((bm, 1), lambda i: (i, 0)),
        out_shape=jax.ShapeDtypeStruct((n, 1), f32),
    )(agg2, hs2, dis, b2, W3, b3, W4, b4)

    return out


# speedup vs baseline: 22.5588x; 1.0569x over previous
import functools

import jax
import jax.numpy as jnp
from jax import lax
from jax.experimental import pallas as pl
from jax.experimental.pallas import tpu as pltpu
from jax.experimental.pallas import tpu_sc as plsc

NC = 2
NS = 16
NW = NC * NS
KB = 128
UNROLL = 6


def _make_deg_kernel(n, nb):
    per_w = nb // NW
    rem = nb - per_w * NW
    chunk = -(-((n + NS - 1) // NS) // 128) * 128
    n_pad = NS * chunk

    mesh = plsc.VectorSubcoreMesh(core_axis_name="c", subcore_axis_name="s")

    @functools.partial(
        pl.kernel,
        mesh=mesh,
        out_type=jax.ShapeDtypeStruct((NC, n_pad), jnp.float32),
        scratch_types=[
            pltpu.VMEM((KB,), jnp.int32),
            pltpu.VMEM((KB,), jnp.float32),
            pltpu.VMEM((chunk,), jnp.float32),
            pltpu.VMEM_SHARED((n_pad,), jnp.float32),
        ],
    )
    def deg_kernel(dst2_hbm, out_hbm, idx_v, ones_v, z_v, hist):
        c = lax.axis_index("c")
        s = lax.axis_index("s")
        w = s * NC + c

        def fill(i, _):
            ones_v[pl.ds(i * 16, 16)] = jnp.full((16,), 1.0, jnp.float32)
            return _

        lax.fori_loop(0, KB // 16, fill, None)

        def zfill(i, _):
            z_v[pl.ds(i * 16, 16)] = jnp.zeros((16,), jnp.float32)
            return _

        lax.fori_loop(0, chunk // 16, zfill, None)
        pltpu.sync_copy(z_v, hist.at[pl.ds(s * chunk, chunk)])
        plsc.subcore_barrier()

        def body(i, _):
            bj = w * per_w + i
            pltpu.sync_copy(dst2_hbm.at[bj], idx_v)
            pltpu.sync_copy(ones_v, hist.at[idx_v], add=True)
            return _

        lax.fori_loop(0, per_w, body, None)
        if rem:
            @pl.when(w < rem)
            def _():
                bj = NW * per_w + w
                pltpu.sync_copy(dst2_hbm.at[bj], idx_v)
                pltpu.sync_copy(ones_v, hist.at[idx_v], add=True)

        plsc.subcore_barrier()
        pltpu.sync_copy(hist.at[pl.ds(s * chunk, chunk)],
                        out_hbm.at[c, pl.ds(s * chunk, chunk)])

    return deg_kernel


def _make_agg_kernel(n, d, nb):
    per_w = nb // NW
    rem = nb - per_w * NW
    rows_per_tile = -(-((n + NS - 1) // NS) // 8) * 8
    n_pad = NS * rows_per_tile
    U = 3
    n_groups = per_w // U
    tail = per_w - n_groups * U

    mesh = plsc.VectorSubcoreMesh(core_axis_name="c", subcore_axis_name="s")

    @functools.partial(
        pl.kernel,
        mesh=mesh,
        out_type=jax.ShapeDtypeStruct((NC, n_pad, d), jnp.float32),
        scratch_types=[
            pltpu.VMEM((KB,), jnp.int32),
            pltpu.VMEM((KB,), jnp.int32),
            pltpu.VMEM((KB,), jnp.int32),
            pltpu.VMEM((KB,), jnp.int32),
            pltpu.VMEM((KB,), jnp.int32),
            pltpu.VMEM((KB,), jnp.int32),
            pltpu.VMEM((KB, d), jnp.float32),
            pltpu.VMEM((KB, d), jnp.float32),
            pltpu.VMEM((KB, d), jnp.float32),
            pltpu.VMEM_SHARED((n_pad, d), jnp.float32),
            pltpu.SemaphoreType.DMA,
            pltpu.SemaphoreType.DMA,
            pltpu.SemaphoreType.DMA,
            pltpu.SemaphoreType.DMA,
            pltpu.SemaphoreType.DMA,
            pltpu.SemaphoreType.DMA,
            pltpu.SemaphoreType.DMA,
            pltpu.SemaphoreType.DMA,
            pltpu.SemaphoreType.DMA,
            pltpu.SemaphoreType.DMA,
            pltpu.SemaphoreType.DMA,
            pltpu.SemaphoreType.DMA,
        ],
    )
    def agg_kernel(hs_hbm, src2_hbm, dst2_hbm, out_hbm,
                   ia0, ia1, ia2,
                   ib0, ib1, ib2,
                   r0_, r1_, r2_,
                   acc,
                   sa0, sa1, sa2,
                   sb0, sb1, sb2,
                   sg0, sg1, sg2,
                   sc0, sc1, sc2):
        idx_s = (ia0, ia1, ia2)
        idx_d = (ib0, ib1, ib2)
        rows = (r0_, r1_, r2_)
        sem_is = (sa0, sa1, sa2)
        sem_id = (sb0, sb1, sb2)
        sem_g = (sg0, sg1, sg2)
        sem_sc = (sc0, sc1, sc2)
        c = lax.axis_index("c")
        s = lax.axis_index("s")
        w = s * NC + c

        z_v = rows[0]

        def zrow(i, _):
            for g in range(d // 16):
                z_v[i, pl.ds(g * 16, 16)] = jnp.zeros((16,), jnp.float32)
            return _

        lax.fori_loop(0, KB, zrow, None)
        r0 = s * rows_per_tile
        off = 0
        while off < rows_per_tile:
            sz = min(KB, rows_per_tile - off)
            pltpu.sync_copy(z_v.at[pl.ds(0, sz)], acc.at[pl.ds(r0 + off, sz)])
            off += sz
        plsc.subcore_barrier()

        def do_block(bj):
            pltpu.sync_copy(src2_hbm.at[bj], idx_s[0])
            pltpu.sync_copy(dst2_hbm.at[bj], idx_d[0])
            pltpu.async_copy(hs_hbm.at[idx_s[0]], rows[0], sem_g[0]).wait()
            pltpu.sync_copy(rows[0], acc.at[idx_d[0]], add=True)

        def body(j, _):
            base = w * per_w + U * j
            d_is = [pltpu.async_copy(src2_hbm.at[base + k], idx_s[k],
                                     sem_is[k]) for k in range(U)]
            d_id = [pltpu.async_copy(dst2_hbm.at[base + k], idx_d[k],
                                     sem_id[k]) for k in range(U)]
            d_g = []
            for k in range(U):
                d_is[k].wait()
                d_g.append(pltpu.async_copy(hs_hbm.at[idx_s[k]], rows[k],
                                            sem_g[k]))
            d_sc = []
            for k in range(U):
                d_g[k].wait()
                d_id[k].wait()
                d_sc.append(pltpu.async_copy(rows[k], acc.at[idx_d[k]],
                                             sem_sc[k], add=True))
            for k in range(U):
                d_sc[k].wait()
            return _

        lax.fori_loop(0, n_groups, body, None)
        for t in range(tail):
            do_block(w * per_w + n_groups * U + t)
        if rem:
            @pl.when(w < rem)
            def _():
                do_block(NW * per_w + w)

        plsc.subcore_barrier()
        lo = s * rows_per_tile
        pltpu.sync_copy(acc.at[pl.ds(lo, rows_per_tile)],
                        out_hbm.at[c, pl.ds(lo, rows_per_tile)])

    return agg_kernel


def _scale_mm(degT_ref, x_ref, w_ref, hs_ref, dis_ref):
    dis = lax.rsqrt(1.0 + degT_ref[:, 0:1] + degT_ref[:, 1:2])
    hs_ref[...] = dis * jnp.dot(x_ref[...], w_ref[...],
                                preferred_element_type=jnp.float32)
    dis_ref[...] = dis


def _mid_layer(agg_ref, hs_ref, dis_ref, b1_ref, w2_ref, o_ref):
    a = agg_ref[0] + agg_ref[1] + hs_ref[...]
    h1 = jnp.tanh(dis_ref[...] * a + b1_ref[...])
    o_ref[...] = dis_ref[...] * jnp.dot(h1, w2_ref[...],
                                        preferred_element_type=jnp.float32)


def _final_head(agg_ref, hs_ref, dis_ref, b2_ref, w3_ref, b3_ref, w4_ref,
                b4_ref, o_ref):
    a = agg_ref[0] + agg_ref[1] + hs_ref[...]
    h2 = jnp.tanh(dis_ref[...] * a + b2_ref[...])
    h3 = jnp.tanh(jnp.dot(h2, w3_ref[...], preferred_element_type=jnp.float32)
                  + b3_ref[...])
    o_ref[...] = jnp.dot(h3, w4_ref[...],
                         preferred_element_type=jnp.float32) + b4_ref[...]


def kernel(x, edge_index, W1, b1, W2, b2, W3, b3, W4, b4):
    n, d = x.shape
    e = edge_index.shape[1]
    nb = e // KB
    assert nb * KB == e

    src2 = edge_index[0].reshape(nb, KB)
    dst2 = edge_index[1].reshape(nb, KB)

    deg_kernel = _make_deg_kernel(n, nb)
    agg_kernel = _make_agg_kernel(n, d, nb)

    degp = deg_kernel(dst2)[:, :n]
    degT = degp.T

    bm = 1000
    grid = (n // bm,)
    f32 = jnp.float32

    hs1, dis = pl.pallas_call(
        _scale_mm,
        grid=grid,
        in_specs=[
            pl.BlockSpec((bm, 2), lambda i: (i, 0)),
            pl.BlockSpec((bm, d), lambda i: (i, 0)),
            pl.BlockSpec((d, d), lambda i: (0, 0)),
        ],
        out_specs=[
            pl.BlockSpec((bm, d), lambda i: (i, 0)),
            pl.BlockSpec((bm, 1), lambda i: (i, 0)),
        ],
        out_shape=[
            jax.ShapeDtypeStruct((n, d), f32),
            jax.ShapeDtypeStruct((n, 1), f32),
        ],
    )(degT, x, W1)

    agg1 = agg_kernel(hs1, src2, dst2)[:, :n]

    hs2 = pl.pallas_call(
        _mid_layer,
        grid=grid,
        in_specs=[
            pl.BlockSpec((2, bm, d), lambda i: (0, i, 0)),
            pl.BlockSpec((bm, d), lambda i: (i, 0)),
            pl.BlockSpec((bm, 1), lambda i: (i, 0)),
            pl.BlockSpec((d,), lambda i: (0,)),
            pl.BlockSpec((d, d), lambda i: (0, 0)),
        ],
        out_specs=pl.BlockSpec((bm, d), lambda i: (i, 0)),
        out_shape=jax.ShapeDtypeStruct((n, d), f32),
    )(agg1, hs1, dis, b1, W2)

    agg2 = agg_kernel(hs2, src2, dst2)[:, :n]

    d3 = W3.shape[1]
    out = pl.pallas_call(
        _final_head,
        grid=grid,
        in_specs=[
            pl.BlockSpec((2, bm, d), lambda i: (0, i, 0)),
            pl.BlockSpec((bm, d), lambda i: (i, 0)),
            pl.BlockSpec((bm, 1), lambda i: (i, 0)),
            pl.BlockSpec((d,), lambda i: (0,)),
            pl.BlockSpec((d, d3), lambda i: (0, 0)),
            pl.BlockSpec((d3,), lambda i: (0,)),
            pl.BlockSpec((d3, 1), lambda i: (0, 0)),
            pl.BlockSpec((1,), lambda i: (0,)),
        ],
        out_specs=pl.BlockSpec((bm, 1), lambda i: (i, 0)),
        out_shape=jax.ShapeDtypeStruct((n, 1), f32),
    )(agg2, hs2, dis, b2, W3, b3, W4, b4)

    return out


# speedup vs baseline: 24.0422x; 1.0658x over previous
import functools

import jax
import jax.numpy as jnp
from jax import lax
from jax.experimental import pallas as pl
from jax.experimental.pallas import tpu as pltpu
from jax.experimental.pallas import tpu_sc as plsc

NC = 2
NS = 16
NW = NC * NS
KB = 128
UNROLL = 6


def _make_deg_kernel(n, nb):
    per_w = nb // NW
    rem = nb - per_w * NW
    chunk = -(-((n + NS - 1) // NS) // 128) * 128
    n_pad = NS * chunk
    U = 3
    n_groups = per_w // U
    tail = per_w - n_groups * U

    mesh = plsc.VectorSubcoreMesh(core_axis_name="c", subcore_axis_name="s")

    @functools.partial(
        pl.kernel,
        mesh=mesh,
        out_type=jax.ShapeDtypeStruct((NC, n_pad), jnp.float32),
        scratch_types=[
            pltpu.VMEM((KB,), jnp.int32),
            pltpu.VMEM((KB,), jnp.int32),
            pltpu.VMEM((KB,), jnp.int32),
            pltpu.VMEM((KB,), jnp.float32),
            pltpu.VMEM((chunk,), jnp.float32),
            pltpu.VMEM_SHARED((n_pad,), jnp.float32),
            pltpu.SemaphoreType.DMA,
            pltpu.SemaphoreType.DMA,
            pltpu.SemaphoreType.DMA,
            pltpu.SemaphoreType.DMA,
            pltpu.SemaphoreType.DMA,
            pltpu.SemaphoreType.DMA,
        ],
    )
    def deg_kernel(dst2_hbm, out_hbm, i0, i1, i2, ones_v, z_v, hist,
                   si0, si1, si2, ss0, ss1, ss2):
        idx = (i0, i1, i2)
        sem_i = (si0, si1, si2)
        sem_s = (ss0, ss1, ss2)
        c = lax.axis_index("c")
        s = lax.axis_index("s")
        w = s * NC + c

        def fill(i, _):
            ones_v[pl.ds(i * 16, 16)] = jnp.full((16,), 1.0, jnp.float32)
            return _

        lax.fori_loop(0, KB // 16, fill, None)

        def zfill(i, _):
            z_v[pl.ds(i * 16, 16)] = jnp.zeros((16,), jnp.float32)
            return _

        lax.fori_loop(0, chunk // 16, zfill, None)
        pltpu.sync_copy(z_v, hist.at[pl.ds(s * chunk, chunk)])
        plsc.subcore_barrier()

        def body(j, _):
            base = w * per_w + 3 * j
            d_i = [pltpu.async_copy(dst2_hbm.at[base + k], idx[k], sem_i[k])
                   for k in range(3)]
            d_s = []
            for k in range(3):
                d_i[k].wait()
                d_s.append(pltpu.async_copy(ones_v, hist.at[idx[k]],
                                            sem_s[k], add=True))
            for k in range(3):
                d_s[k].wait()
            return _

        lax.fori_loop(0, n_groups, body, None)
        for t in range(tail):
            bj = w * per_w + n_groups * U + t
            pltpu.sync_copy(dst2_hbm.at[bj], idx[0])
            pltpu.sync_copy(ones_v, hist.at[idx[0]], add=True)
        if rem:
            @pl.when(w < rem)
            def _():
                bj = NW * per_w + w
                pltpu.sync_copy(dst2_hbm.at[bj], idx[0])
                pltpu.sync_copy(ones_v, hist.at[idx[0]], add=True)

        plsc.subcore_barrier()
        pltpu.sync_copy(hist.at[pl.ds(s * chunk, chunk)],
                        out_hbm.at[c, pl.ds(s * chunk, chunk)])

    return deg_kernel


def _make_agg_kernel(n, d, nb):
    per_w = nb // NW
    rem = nb - per_w * NW
    rows_per_tile = -(-((n + NS - 1) // NS) // 8) * 8
    n_pad = NS * rows_per_tile
    U = 3
    n_groups = per_w // U
    tail = per_w - n_groups * U

    mesh = plsc.VectorSubcoreMesh(core_axis_name="c", subcore_axis_name="s")

    @functools.partial(
        pl.kernel,
        mesh=mesh,
        out_type=jax.ShapeDtypeStruct((NC, n_pad, d), jnp.float32),
        scratch_types=[
            pltpu.VMEM((KB,), jnp.int32),
            pltpu.VMEM((KB,), jnp.int32),
            pltpu.VMEM((KB,), jnp.int32),
            pltpu.VMEM((KB,), jnp.int32),
            pltpu.VMEM((KB,), jnp.int32),
            pltpu.VMEM((KB,), jnp.int32),
            pltpu.VMEM((KB, d), jnp.float32),
            pltpu.VMEM((KB, d), jnp.float32),
            pltpu.VMEM((KB, d), jnp.float32),
            pltpu.VMEM_SHARED((n_pad, d), jnp.float32),
            pltpu.SemaphoreType.DMA,
            pltpu.SemaphoreType.DMA,
            pltpu.SemaphoreType.DMA,
            pltpu.SemaphoreType.DMA,
            pltpu.SemaphoreType.DMA,
            pltpu.SemaphoreType.DMA,
            pltpu.SemaphoreType.DMA,
            pltpu.SemaphoreType.DMA,
            pltpu.SemaphoreType.DMA,
            pltpu.SemaphoreType.DMA,
            pltpu.SemaphoreType.DMA,
            pltpu.SemaphoreType.DMA,
        ],
    )
    def agg_kernel(hs_hbm, src2_hbm, dst2_hbm, out_hbm,
                   ia0, ia1, ia2,
                   ib0, ib1, ib2,
                   r0_, r1_, r2_,
                   acc,
                   sa0, sa1, sa2,
                   sb0, sb1, sb2,
                   sg0, sg1, sg2,
                   sc0, sc1, sc2):
        idx_s = (ia0, ia1, ia2)
        idx_d = (ib0, ib1, ib2)
        rows = (r0_, r1_, r2_)
        sem_is = (sa0, sa1, sa2)
        sem_id = (sb0, sb1, sb2)
        sem_g = (sg0, sg1, sg2)
        sem_sc = (sc0, sc1, sc2)
        c = lax.axis_index("c")
        s = lax.axis_index("s")
        w = s * NC + c

        z_v = rows[0]

        def zrow(i, _):
            for g in range(d // 16):
                z_v[i, pl.ds(g * 16, 16)] = jnp.zeros((16,), jnp.float32)
            return _

        lax.fori_loop(0, KB, zrow, None)
        r0 = s * rows_per_tile
        off = 0
        while off < rows_per_tile:
            sz = min(KB, rows_per_tile - off)
            pltpu.sync_copy(z_v.at[pl.ds(0, sz)], acc.at[pl.ds(r0 + off, sz)])
            off += sz
        plsc.subcore_barrier()

        def do_block(bj):
            pltpu.sync_copy(src2_hbm.at[bj], idx_s[0])
            pltpu.sync_copy(dst2_hbm.at[bj], idx_d[0])
            pltpu.async_copy(hs_hbm.at[idx_s[0]], rows[0], sem_g[0]).wait()
            pltpu.sync_copy(rows[0], acc.at[idx_d[0]], add=True)

        def body(j, _):
            base = w * per_w + U * j
            d_is = [pltpu.async_copy(src2_hbm.at[base + k], idx_s[k],
                                     sem_is[k]) for k in range(U)]
            d_id = [pltpu.async_copy(dst2_hbm.at[base + k], idx_d[k],
                                     sem_id[k]) for k in range(U)]
            d_g = []
            for k in range(U):
                d_is[k].wait()
                d_g.append(pltpu.async_copy(hs_hbm.at[idx_s[k]], rows[k],
                                            sem_g[k]))
            d_sc = []
            for k in range(U):
                d_g[k].wait()
                d_id[k].wait()
                d_sc.append(pltpu.async_copy(rows[k], acc.at[idx_d[k]],
                                             sem_sc[k], add=True))
            for k in range(U):
                d_sc[k].wait()
            return _

        lax.fori_loop(0, n_groups, body, None)
        for t in range(tail):
            do_block(w * per_w + n_groups * U + t)
        if rem:
            @pl.when(w < rem)
            def _():
                do_block(NW * per_w + w)

        plsc.subcore_barrier()
        lo = s * rows_per_tile
        pltpu.sync_copy(acc.at[pl.ds(lo, rows_per_tile)],
                        out_hbm.at[c, pl.ds(lo, rows_per_tile)])

    return agg_kernel


def _scale_mm(degT_ref, x_ref, w_ref, hs_ref, dis_ref):
    dis = lax.rsqrt(1.0 + degT_ref[:, 0:1] + degT_ref[:, 1:2])
    hs_ref[...] = dis * jnp.dot(x_ref[...], w_ref[...],
                                preferred_element_type=jnp.float32)
    dis_ref[...] = dis


def _mid_layer(agg_ref, hs_ref, dis_ref, b1_ref, w2_ref, o_ref):
    a = agg_ref[0] + agg_ref[1] + hs_ref[...]
    h1 = jnp.tanh(dis_ref[...] * a + b1_ref[...])
    o_ref[...] = dis_ref[...] * jnp.dot(h1, w2_ref[...],
                                        preferred_element_type=jnp.float32)


def _final_head(agg_ref, hs_ref, dis_ref, b2_ref, w3_ref, b3_ref, w4_ref,
                b4_ref, o_ref):
    a = agg_ref[0] + agg_ref[1] + hs_ref[...]
    h2 = jnp.tanh(dis_ref[...] * a + b2_ref[...])
    h3 = jnp.tanh(jnp.dot(h2, w3_ref[...], preferred_element_type=jnp.float32)
                  + b3_ref[...])
    o_ref[...] = jnp.dot(h3, w4_ref[...],
                         preferred_element_type=jnp.float32) + b4_ref[...]


def kernel(x, edge_index, W1, b1, W2, b2, W3, b3, W4, b4):
    n, d = x.shape
    e = edge_index.shape[1]
    nb = e // KB
    assert nb * KB == e

    src2 = edge_index[0].reshape(nb, KB)
    dst2 = edge_index[1].reshape(nb, KB)

    deg_kernel = _make_deg_kernel(n, nb)
    agg_kernel = _make_agg_kernel(n, d, nb)

    degp = deg_kernel(dst2)[:, :n]
    degT = degp.T

    bm = 1000
    grid = (n // bm,)
    f32 = jnp.float32

    hs1, dis = pl.pallas_call(
        _scale_mm,
        grid=grid,
        in_specs=[
            pl.BlockSpec((bm, 2), lambda i: (i, 0)),
            pl.BlockSpec((bm, d), lambda i: (i, 0)),
            pl.BlockSpec((d, d), lambda i: (0, 0)),
        ],
        out_specs=[
            pl.BlockSpec((bm, d), lambda i: (i, 0)),
            pl.BlockSpec((bm, 1), lambda i: (i, 0)),
        ],
        out_shape=[
            jax.ShapeDtypeStruct((n, d), f32),
            jax.ShapeDtypeStruct((n, 1), f32),
        ],
    )(degT, x, W1)

    agg1 = agg_kernel(hs1, src2, dst2)[:, :n]

    hs2 = pl.pallas_call(
        _mid_layer,
        grid=grid,
        in_specs=[
            pl.BlockSpec((2, bm, d), lambda i: (0, i, 0)),
            pl.BlockSpec((bm, d), lambda i: (i, 0)),
            pl.BlockSpec((bm, 1), lambda i: (i, 0)),
            pl.BlockSpec((d,), lambda i: (0,)),
            pl.BlockSpec((d, d), lambda i: (0, 0)),
        ],
        out_specs=pl.BlockSpec((bm, d), lambda i: (i, 0)),
        out_shape=jax.ShapeDtypeStruct((n, d), f32),
    )(agg1, hs1, dis, b1, W2)

    agg2 = agg_kernel(hs2, src2, dst2)[:, :n]

    d3 = W3.shape[1]
    out = pl.pallas_call(
        _final_head,
        grid=grid,
        in_specs=[
            pl.BlockSpec((2, bm, d), lambda i: (0, i, 0)),
            pl.BlockSpec((bm, d), lambda i: (i, 0)),
            pl.BlockSpec((bm, 1), lambda i: (i, 0)),
            pl.BlockSpec((d,), lambda i: (0,)),
            pl.BlockSpec((d, d3), lambda i: (0, 0)),
            pl.BlockSpec((d3,), lambda i: (0,)),
            pl.BlockSpec((d3, 1), lambda i: (0, 0)),
            pl.BlockSpec((1,), lambda i: (0,)),
        ],
        out_specs=pl.BlockSpec((bm, 1), lambda i: (i, 0)),
        out_shape=jax.ShapeDtypeStruct((n, 1), f32),
    )(agg2, hs2, dis, b2, W3, b3, W4, b4)

    return out


# speedup vs baseline: 24.9118x; 1.0362x over previous
import functools

import jax
import jax.numpy as jnp
from jax import lax
from jax.experimental import pallas as pl
from jax.experimental.pallas import tpu as pltpu
from jax.experimental.pallas import tpu_sc as plsc

NC = 2
NS = 16
NW = NC * NS
KB = 128
UNROLL = 6


def _make_deg_kernel(n, nb):
    per_w = nb // NW
    rem = nb - per_w * NW
    chunk = -(-((n + NS - 1) // NS) // 128) * 128
    n_pad = NS * chunk
    U = 3
    n_groups = per_w // U
    tail = per_w - n_groups * U

    mesh = plsc.VectorSubcoreMesh(core_axis_name="c", subcore_axis_name="s")

    @functools.partial(
        pl.kernel,
        mesh=mesh,
        out_type=jax.ShapeDtypeStruct((NC, n_pad), jnp.float32),
        scratch_types=[
            pltpu.VMEM((KB,), jnp.int32),
            pltpu.VMEM((KB,), jnp.int32),
            pltpu.VMEM((KB,), jnp.int32),
            pltpu.VMEM((KB,), jnp.float32),
            pltpu.VMEM((chunk,), jnp.float32),
            pltpu.VMEM_SHARED((n_pad,), jnp.float32),
            pltpu.SemaphoreType.DMA,
            pltpu.SemaphoreType.DMA,
            pltpu.SemaphoreType.DMA,
            pltpu.SemaphoreType.DMA,
            pltpu.SemaphoreType.DMA,
            pltpu.SemaphoreType.DMA,
        ],
    )
    def deg_kernel(dst2_hbm, out_hbm, i0, i1, i2, ones_v, z_v, hist,
                   si0, si1, si2, ss0, ss1, ss2):
        idx = (i0, i1, i2)
        sem_i = (si0, si1, si2)
        sem_s = (ss0, ss1, ss2)
        c = lax.axis_index("c")
        s = lax.axis_index("s")
        w = s * NC + c

        def fill(i, _):
            ones_v[pl.ds(i * 16, 16)] = jnp.full((16,), 1.0, jnp.float32)
            return _

        lax.fori_loop(0, KB // 16, fill, None)

        def zfill(i, _):
            z_v[pl.ds(i * 16, 16)] = jnp.zeros((16,), jnp.float32)
            return _

        lax.fori_loop(0, chunk // 16, zfill, None)
        pltpu.sync_copy(z_v, hist.at[pl.ds(s * chunk, chunk)])
        plsc.subcore_barrier()

        def body(j, _):
            base = w * per_w + 3 * j
            d_i = [pltpu.async_copy(dst2_hbm.at[base + k], idx[k], sem_i[k])
                   for k in range(3)]
            d_s = []
            for k in range(3):
                d_i[k].wait()
                d_s.append(pltpu.async_copy(ones_v, hist.at[idx[k]],
                                            sem_s[k], add=True))
            for k in range(3):
                d_s[k].wait()
            return _

        lax.fori_loop(0, n_groups, body, None)
        for t in range(tail):
            bj = w * per_w + n_groups * U + t
            pltpu.sync_copy(dst2_hbm.at[bj], idx[0])
            pltpu.sync_copy(ones_v, hist.at[idx[0]], add=True)
        if rem:
            @pl.when(w < rem)
            def _():
                bj = NW * per_w + w
                pltpu.sync_copy(dst2_hbm.at[bj], idx[0])
                pltpu.sync_copy(ones_v, hist.at[idx[0]], add=True)

        plsc.subcore_barrier()
        pltpu.sync_copy(hist.at[pl.ds(s * chunk, chunk)],
                        out_hbm.at[c, pl.ds(s * chunk, chunk)])

    return deg_kernel


def _make_agg_kernel(n, d, nb):
    per_w = nb // NW
    rem = nb - per_w * NW
    rows_per_tile = -(-((n + NS - 1) // NS) // 8) * 8
    n_pad = NS * rows_per_tile
    U = 3
    n_groups = per_w // U
    tail = per_w - n_groups * U

    mesh = plsc.VectorSubcoreMesh(core_axis_name="c", subcore_axis_name="s")

    @functools.partial(
        pl.kernel,
        mesh=mesh,
        out_type=jax.ShapeDtypeStruct((NC, n_pad, d), jnp.float32),
        scratch_types=[
            pltpu.VMEM((KB,), jnp.int32),
            pltpu.VMEM((KB,), jnp.int32),
            pltpu.VMEM((KB,), jnp.int32),
            pltpu.VMEM((KB,), jnp.int32),
            pltpu.VMEM((KB,), jnp.int32),
            pltpu.VMEM((KB,), jnp.int32),
            pltpu.VMEM((KB, d), jnp.float32),
            pltpu.VMEM((KB, d), jnp.float32),
            pltpu.VMEM((KB, d), jnp.float32),
            pltpu.VMEM_SHARED((n_pad, d), jnp.float32),
            pltpu.SemaphoreType.DMA,
            pltpu.SemaphoreType.DMA,
            pltpu.SemaphoreType.DMA,
            pltpu.SemaphoreType.DMA,
            pltpu.SemaphoreType.DMA,
            pltpu.SemaphoreType.DMA,
            pltpu.SemaphoreType.DMA,
            pltpu.SemaphoreType.DMA,
            pltpu.SemaphoreType.DMA,
            pltpu.SemaphoreType.DMA,
            pltpu.SemaphoreType.DMA,
            pltpu.SemaphoreType.DMA,
        ],
    )
    def agg_kernel(hs_hbm, src2_hbm, dst2_hbm, out_hbm,
                   ia0, ia1, ia2,
                   ib0, ib1, ib2,
                   r0_, r1_, r2_,
                   acc,
                   sa0, sa1, sa2,
                   sb0, sb1, sb2,
                   sg0, sg1, sg2,
                   sc0, sc1, sc2):
        idx_s = (ia0, ia1, ia2)
        idx_d = (ib0, ib1, ib2)
        rows = (r0_, r1_, r2_)
        sem_is = (sa0, sa1, sa2)
        sem_id = (sb0, sb1, sb2)
        sem_g = (sg0, sg1, sg2)
        sem_sc = (sc0, sc1, sc2)
        c = lax.axis_index("c")
        s = lax.axis_index("s")
        w = s * NC + c

        z_v = rows[0]

        def zrow(i, _):
            for g in range(d // 16):
                z_v[i, pl.ds(g * 16, 16)] = jnp.zeros((16,), jnp.float32)
            return _

        lax.fori_loop(0, KB, zrow, None)
        r0 = s * rows_per_tile
        off = 0
        while off < rows_per_tile:
            sz = min(KB, rows_per_tile - off)
            pltpu.sync_copy(z_v.at[pl.ds(0, sz)], acc.at[pl.ds(r0 + off, sz)])
            off += sz
        plsc.subcore_barrier()

        def do_block(bj):
            pltpu.sync_copy(src2_hbm.at[bj], idx_s[0])
            pltpu.sync_copy(dst2_hbm.at[bj], idx_d[0])
            pltpu.async_copy(hs_hbm.at[idx_s[0]], rows[0], sem_g[0]).wait()
            pltpu.sync_copy(rows[0], acc.at[idx_d[0]], add=True)

        def body(j, _):
            base = w * per_w + U * j
            d_is = [pltpu.async_copy(src2_hbm.at[base + k], idx_s[k],
                                     sem_is[k]) for k in range(U)]
            d_id = [pltpu.async_copy(dst2_hbm.at[base + k], idx_d[k],
                                     sem_id[k]) for k in range(U)]
            d_g = []
            for k in range(U):
                d_is[k].wait()
                d_g.append(pltpu.async_copy(hs_hbm.at[idx_s[k]], rows[k],
                                            sem_g[k]))
            d_sc = []
            for k in range(U):
                d_g[k].wait()
                d_id[k].wait()
                d_sc.append(pltpu.async_copy(rows[k], acc.at[idx_d[k]],
                                             sem_sc[k], add=True))
            for k in range(U):
                d_sc[k].wait()
            return _

        lax.fori_loop(0, n_groups, body, None)
        for t in range(tail):
            do_block(w * per_w + n_groups * U + t)
        if rem:
            @pl.when(w < rem)
            def _():
                do_block(NW * per_w + w)

        plsc.subcore_barrier()
        lo = s * rows_per_tile
        pltpu.sync_copy(acc.at[pl.ds(lo, rows_per_tile)],
                        out_hbm.at[c, pl.ds(lo, rows_per_tile)])

    return agg_kernel


def _scale_mm(degT_ref, x_ref, w_ref, hs_ref, dis_ref):
    dis = lax.rsqrt(1.0 + degT_ref[:, 0:1] + degT_ref[:, 1:2])
    hs_ref[...] = dis * jnp.dot(x_ref[...], w_ref[...],
                                preferred_element_type=jnp.float32)
    dis_ref[...] = dis


def _mid_layer(agg_ref, hs_ref, dis_ref, b1_ref, w2_ref, o_ref):
    a = agg_ref[0] + agg_ref[1] + hs_ref[...]
    h1 = jnp.tanh(dis_ref[...] * a + b1_ref[...])
    o_ref[...] = dis_ref[...] * jnp.dot(h1, w2_ref[...],
                                        preferred_element_type=jnp.float32)


def _final_head(agg_ref, hs_ref, dis_ref, b2_ref, w3_ref, b3_ref, w4_ref,
                b4_ref, o_ref):
    a = agg_ref[0] + agg_ref[1] + hs_ref[...]
    h2 = jnp.tanh(dis_ref[...] * a + b2_ref[...])
    h3 = jnp.tanh(jnp.dot(h2, w3_ref[...], preferred_element_type=jnp.float32)
                  + b3_ref[...])
    o_ref[...] = jnp.dot(h3, w4_ref[...],
                         preferred_element_type=jnp.float32) + b4_ref[...]


def kernel(x, edge_index, W1, b1, W2, b2, W3, b3, W4, b4):
    n, d = x.shape
    e = edge_index.shape[1]
    nb = e // KB
    assert nb * KB == e

    src2 = edge_index[0].reshape(nb, KB)
    dst2 = edge_index[1].reshape(nb, KB)

    deg_kernel = _make_deg_kernel(n, nb)
    agg_kernel = _make_agg_kernel(n, d, nb)

    degT = deg_kernel(dst2).T

    bm = 1000
    grid = (n // bm,)
    f32 = jnp.float32

    hs1, dis = pl.pallas_call(
        _scale_mm,
        grid=grid,
        in_specs=[
            pl.BlockSpec((bm, 2), lambda i: (i, 0)),
            pl.BlockSpec((bm, d), lambda i: (i, 0)),
            pl.BlockSpec((d, d), lambda i: (0, 0)),
        ],
        out_specs=[
            pl.BlockSpec((bm, d), lambda i: (i, 0)),
            pl.BlockSpec((bm, 1), lambda i: (i, 0)),
        ],
        out_shape=[
            jax.ShapeDtypeStruct((n, d), f32),
            jax.ShapeDtypeStruct((n, 1), f32),
        ],
    )(degT, x, W1)

    agg1 = agg_kernel(hs1, src2, dst2)

    hs2 = pl.pallas_call(
        _mid_layer,
        grid=grid,
        in_specs=[
            pl.BlockSpec((2, bm, d), lambda i: (0, i, 0)),
            pl.BlockSpec((bm, d), lambda i: (i, 0)),
            pl.BlockSpec((bm, 1), lambda i: (i, 0)),
            pl.BlockSpec((d,), lambda i: (0,)),
            pl.BlockSpec((d, d), lambda i: (0, 0)),
        ],
        out_specs=pl.BlockSpec((bm, d), lambda i: (i, 0)),
        out_shape=jax.ShapeDtypeStruct((n, d), f32),
    )(agg1, hs1, dis, b1, W2)

    agg2 = agg_kernel(hs2, src2, dst2)

    d3 = W3.shape[1]
    out = pl.pallas_call(
        _final_head,
        grid=grid,
        in_specs=[
            pl.BlockSpec((2, bm, d), lambda i: (0, i, 0)),
            pl.BlockSpec((bm, d), lambda i: (i, 0)),
            pl.BlockSpec((bm, 1), lambda i: (i, 0)),
            pl.BlockSpec((d,), lambda i: (0,)),
            pl.BlockSpec((d, d3), lambda i: (0, 0)),
            pl.BlockSpec((d3,), lambda i: (0,)),
            pl.BlockSpec((d3, 1), lambda i: (0, 0)),
            pl.BlockSpec((1,), lambda i: (0,)),
        ],
        out_specs=pl.BlockSpec((bm, 1), lambda i: (i, 0)),
        out_shape=jax.ShapeDtypeStruct((n, 1), f32),
    )(agg2, hs2, dis, b2, W3, b3, W4, b4)

    return out
